# baseline (speedup 1.0000x reference)
import functools

import jax
import jax.numpy as jnp
from jax import lax
from jax.experimental import pallas as pl
from jax.experimental.pallas import tpu as pltpu
from jax.experimental.pallas import tpu_sc as plsc

E = 8
TOPK = 2
N = 8192
D_IN = 4096
D_OUT = 4096
BM = 256
NB = N // BM
NBLK = N * TOPK // BM + E
PAD = NBLK * BM
BN = 1024
NJ = D_OUT // BN

_MESH = dict(core_axis_name="c", subcore_axis_name="s")
NC, NS = 2, 16
NW = NC * NS


def _top2(noisy):
    col = lax.broadcasted_iota(jnp.int32, noisy.shape, 1)
    m1 = jnp.max(noisy, axis=1, keepdims=True)
    a1 = jnp.min(jnp.where(noisy == m1, col, E), axis=1, keepdims=True)
    oh1 = col == a1
    masked = jnp.where(oh1, -jnp.inf, noisy)
    m2 = jnp.max(masked, axis=1, keepdims=True)
    a2 = jnp.min(jnp.where(masked == m2, col, E), axis=1, keepdims=True)
    oh2 = col == a2
    z = jnp.exp(m2 - m1)
    p1 = 1.0 / (1.0 + z)
    p2 = 1.0 - p1
    return oh1, oh2, p1, p2


def _router_body(x_ref, wg_ref, bg_ref, wn_ref, bn_ref, eps_ref,
                 gate_ref, d1_ref, d2_ref, w1_ref, w2_ref, meta_ref,
                 noisy_s, bc_s, cum_s, po_s):
    ph = pl.program_id(0)
    i = pl.program_id(1)

    @pl.when(ph == 0)
    def _phase0():
        xb = x_ref[...]
        logits = jnp.dot(xb, wg_ref[...], preferred_element_type=jnp.float32) + bg_ref[...]
        nl = jnp.dot(xb, wn_ref[...], preferred_element_type=jnp.float32) + bn_ref[...]
        sp = jnp.maximum(nl, 0.0) + jnp.log1p(jnp.exp(-jnp.abs(nl)))
        noisy = logits + eps_ref[...] * sp
        noisy_s[i] = noisy
        oh1, oh2, p1, p2 = _top2(noisy)
        gate_ref[...] = jnp.where(oh1, p1, 0.0) + jnp.where(oh2, p2, 0.0)
        d1_ref[...] = jnp.zeros_like(d1_ref)
        d2_ref[...] = jnp.zeros_like(d2_ref)
        w1_ref[...] = jnp.zeros_like(w1_ref)
        w2_ref[...] = jnp.zeros_like(w2_ref)
        sel = jnp.where(oh1, 1.0, 0.0) + jnp.where(oh2, 1.0, 0.0)
        bc_s[i] = jnp.sum(sel, axis=0, keepdims=True)

        @pl.when(i == 0)
        def _():
            meta_ref[...] = jnp.zeros_like(meta_ref)

    @pl.when(ph == 1)
    def _phase1():
        @pl.when(i == 0)
        def _prefix():
            bc = bc_s[...].reshape(NB, E)
            r32 = lax.broadcasted_iota(jnp.int32, (NB, NB), 0)
            c32 = lax.broadcasted_iota(jnp.int32, (NB, NB), 1)
            l32 = jnp.where(r32 > c32, 1.0, 0.0)
            cum_s[...] = jnp.dot(l32, bc, preferred_element_type=jnp.float32).reshape(NB, 1, E)
            tot = jnp.sum(bc, axis=0, keepdims=True)
            nb = jnp.floor((tot + (BM - 1.0)) / BM)
            r8 = lax.broadcasted_iota(jnp.int32, (E, E), 0)
            c8 = lax.broadcasted_iota(jnp.int32, (E, E), 1)
            u8 = jnp.where(r8 < c8, 1.0, 0.0)
            po_blk = jnp.dot(nb, u8, preferred_element_type=jnp.float32)
            po_s[...] = po_blk * BM
            bi = lax.broadcasted_iota(jnp.int32, (128, E), 0).astype(jnp.float32)
            cmat = jnp.where(bi >= po_blk, 1.0, 0.0)
            ones8 = jnp.ones((E, 1), jnp.float32)
            blk_e = jnp.dot(cmat, ones8, preferred_element_type=jnp.float32) - 1.0
            blk_e = jnp.clip(blk_e, 0.0, E - 1.0)
            totblk = jnp.dot(nb, ones8, preferred_element_type=jnp.float32)
            bi1 = lax.broadcasted_iota(jnp.int32, (128, 1), 0).astype(jnp.float32)
            validb = jnp.where(bi1 < totblk, 1.0, 0.0)
            zer = jnp.zeros((128, E - 2), jnp.float32)
            meta_ref[...] = jnp.concatenate([blk_e, validb, zer], axis=1).astype(jnp.int32)

        noisy = noisy_s[i]
        oh1, oh2, p1, p2 = _top2(noisy)
        gate_ref[...] = jnp.where(oh1, p1, 0.0) + jnp.where(oh2, p2, 0.0)
        sel = jnp.where(oh1, 1.0, 0.0) + jnp.where(oh2, 1.0, 0.0)
        rr = lax.broadcasted_iota(jnp.int32, (BM, BM), 0)
        cc = lax.broadcasted_iota(jnp.int32, (BM, BM), 1)
        ltri = jnp.where(rr > cc, 1.0, 0.0)
        rank = jnp.dot(ltri, sel, preferred_element_type=jnp.float32)
        dest_all = po_s[...] + cum_s[i] + rank
        d1_ref[...] = jnp.sum(jnp.where(oh1, dest_all, 0.0), axis=1, keepdims=True).astype(jnp.int32)
        d2_ref[...] = jnp.sum(jnp.where(oh2, dest_all, 0.0), axis=1, keepdims=True).astype(jnp.int32)
        w1_ref[...] = p1
        w2_ref[...] = p2


def _sc_scatter_body(d1_h, d2_h, w1f_h, w2f_h, tok_h, src_h, ws_h,
                     idx1, idx2, tokv, w1, w2, sem):
    wid = lax.axis_index("s") * NC + lax.axis_index("c")
    tpw = N // NW
    base = wid * tpw
    pltpu.sync_copy(d1_h.at[pl.ds(base, tpw)], idx1)
    pltpu.sync_copy(d2_h.at[pl.ds(base, tpw)], idx2)
    pltpu.sync_copy(w1f_h.at[pl.ds(base, tpw)], w1)
    pltpu.sync_copy(w2f_h.at[pl.ds(base, tpw)], w2)
    pltpu.sync_copy(tok_h.at[pl.ds(base, tpw)], tokv)
    h1 = pltpu.async_copy(tokv, src_h.at[idx1], sem)
    h2 = pltpu.async_copy(tokv, src_h.at[idx2], sem)
    h3 = pltpu.async_copy(w1, ws_h.at[idx1], sem)
    h4 = pltpu.async_copy(w2, ws_h.at[idx2], sem)
    h1.wait()
    h2.wait()
    h3.wait()
    h4.wait()


def _sc_gather_body(src_h, x_h, xg_h, idx_v, bufs0, bufs1, bufs2,
                    gs0, gs1, gs2, ws0, ws1, ws2):
    wid = lax.axis_index("s") * NC + lax.axis_index("c")
    spw = PAD // NW
    base = wid * spw
    bufs = (bufs0, bufs1, bufs2)
    gsem = (gs0, gs1, gs2)
    wsem = (ws0, ws1, ws2)
    pltpu.sync_copy(src_h.at[pl.ds(base, spw)], idx_v)

    def clamp(c, _):
        v = idx_v[pl.ds(c * 16, 16)]
        idx_v[pl.ds(c * 16, 16)] = jnp.minimum(jnp.maximum(v, 0), N - 1)
        return 0

    lax.fori_loop(0, spw // 16, clamp, 0)

    nit = spw // 24

    def gbody(it, _):
        @pl.when(it > 0)
        def _():
            for k in range(3):
                pltpu.make_async_copy(
                    bufs[k], xg_h.at[pl.ds(base + ((it - 1) * 3 + k) * 8, 8)],
                    wsem[k]).wait()

        hs = [pltpu.async_copy(x_h.at[idx_v.at[pl.ds((it * 3 + k) * 8, 8)]],
                               bufs[k], gsem[k]) for k in range(3)]
        for k in range(3):
            hs[k].wait()
            pltpu.async_copy(bufs[k], xg_h.at[pl.ds(base + (it * 3 + k) * 8, 8)],
                             wsem[k])
        return 0

    lax.fori_loop(0, nit, gbody, 0)
    for k in range(3):
        pltpu.make_async_copy(
            bufs[k], xg_h.at[pl.ds(base + ((nit - 1) * 3 + k) * 8, 8)],
            wsem[k]).wait()


def _gmm_body(be_s, bv_s, xg_ref, we_ref, bias_ref, w_ref, out_ref):
    i = pl.program_id(1)

    @pl.when(bv_s[i] == 1)
    def _():
        y = jnp.dot(xg_ref[...], we_ref[0], preferred_element_type=jnp.float32)
        out_ref[...] = (y + bias_ref[0]) * w_ref[...]


def _sc_combine_body(d1_h, d2_h, yg_h, out_h, d1, d2,
                     bufa0, bufa1, bufb, sa0, sa1, sb):
    wid = lax.axis_index("s") * NC + lax.axis_index("c")
    tpw = N // NW
    base = wid * tpw
    pltpu.sync_copy(d1_h.at[pl.ds(base, tpw)], d1)
    pltpu.sync_copy(d2_h.at[pl.ds(base, tpw)], d2)

    nchunk = tpw // 8

    def start1(c, ba, sa):
        pltpu.async_copy(yg_h.at[d1.at[pl.ds(c * 8, 8)]], ba, sa)

    def start2(c):
        pltpu.async_copy(yg_h.at[d2.at[pl.ds(c * 8, 8)]], bufb, sb)

    def finish(c, ba, sa):
        pltpu.make_async_copy(yg_h.at[d1.at[pl.ds(c * 8, 8)]], ba, sa).wait()
        pltpu.make_async_copy(yg_h.at[d2.at[pl.ds(c * 8, 8)]], bufb, sb).wait()
        for r in range(8):
            def add16(t, _, r=r):
                for u in range(16):
                    o = t * 256 + u * 16
                    ba[r, pl.ds(o, 16)] = ba[r, pl.ds(o, 16)] + bufb[r, pl.ds(o, 16)]
                return 0

            lax.fori_loop(0, D_OUT // 256, add16, 0)
        pltpu.sync_copy(ba, out_h.at[pl.ds(base + c * 8, 8)])

    start1(0, bufa0, sa0)
    start2(0)

    def cbody(it, _):
        c0 = it * 2
        start1(c0 + 1, bufa1, sa1)
        finish(c0, bufa0, sa0)
        start2(c0 + 1)

        @pl.when(c0 + 2 < nchunk)
        def _():
            start1(c0 + 2, bufa0, sa0)

        finish(c0 + 1, bufa1, sa1)

        @pl.when(c0 + 2 < nchunk)
        def _():
            start2(c0 + 2)

        return 0

    lax.fori_loop(0, nchunk // 2, cbody, 0)


def kernel(x, Wg, bg, Wn, bn, We, be):
    eps = jax.random.normal(jax.random.key(42), (N, E), jnp.float32)
    tok = jnp.arange(N, dtype=jnp.int32)

    gating, d1a, d2a, w1a, w2a, meta = pl.pallas_call(
        _router_body,
        grid=(2, NB),
        in_specs=[
            pl.BlockSpec((BM, D_IN), lambda ph, i: (i * (1 - ph), 0)),
            pl.BlockSpec((D_IN, E), lambda ph, i: (0, 0)),
            pl.BlockSpec((1, E), lambda ph, i: (0, 0)),
            pl.BlockSpec((D_IN, E), lambda ph, i: (0, 0)),
            pl.BlockSpec((1, E), lambda ph, i: (0, 0)),
            pl.BlockSpec((BM, E), lambda ph, i: (i * (1 - ph), 0)),
        ],
        out_specs=[
            pl.BlockSpec((BM, E), lambda ph, i: (i, 0)),
            pl.BlockSpec((BM, 1), lambda ph, i: (i, 0)),
            pl.BlockSpec((BM, 1), lambda ph, i: (i, 0)),
            pl.BlockSpec((BM, 1), lambda ph, i: (i, 0)),
            pl.BlockSpec((BM, 1), lambda ph, i: (i, 0)),
            pl.BlockSpec((128, E), lambda ph, i: (0, 0)),
        ],
        out_shape=[
            jax.ShapeDtypeStruct((N, E), jnp.float32),
            jax.ShapeDtypeStruct((N, 1), jnp.int32),
            jax.ShapeDtypeStruct((N, 1), jnp.int32),
            jax.ShapeDtypeStruct((N, 1), jnp.float32),
            jax.ShapeDtypeStruct((N, 1), jnp.float32),
            jax.ShapeDtypeStruct((128, E), jnp.int32),
        ],
        scratch_shapes=[
            pltpu.VMEM((NB, BM, E), jnp.float32),
            pltpu.VMEM((NB, 1, E), jnp.float32),
            pltpu.VMEM((NB, 1, E), jnp.float32),
            pltpu.VMEM((1, E), jnp.float32),
        ],
    )(x, Wg, bg.reshape(1, E), Wn, bn.reshape(1, E), eps)

    blk_e = meta[:NBLK, 0]
    blk_v = meta[:NBLK, 1]
    d1f = d1a.reshape(N)
    d2f = d2a.reshape(N)

    tpw = N // NW
    scatter_fn = functools.partial(
        pl.kernel,
        out_type=[
            jax.ShapeDtypeStruct((PAD,), jnp.int32),
            jax.ShapeDtypeStruct((PAD,), jnp.float32),
        ],
        mesh=plsc.VectorSubcoreMesh(**_MESH),
        scratch_types=[
            pltpu.VMEM((tpw,), jnp.int32),
            pltpu.VMEM((tpw,), jnp.int32),
            pltpu.VMEM((tpw,), jnp.int32),
            pltpu.VMEM((tpw,), jnp.float32),
            pltpu.VMEM((tpw,), jnp.float32),
            pltpu.SemaphoreType.DMA,
        ],
    )(_sc_scatter_body)
    src_ids, wsort = scatter_fn(d1f, d2f, w1a.reshape(N), w2a.reshape(N), tok)

    gather_fn = functools.partial(
        pl.kernel,
        out_type=jax.ShapeDtypeStruct((PAD, D_IN), jnp.float32),
        mesh=plsc.VectorSubcoreMesh(**_MESH),
        scratch_types=(
            [pltpu.VMEM((PAD // NW,), jnp.int32)]
            + [pltpu.VMEM((8, D_IN), jnp.float32)] * 3
            + [pltpu.SemaphoreType.DMA] * 6
        ),
    )(_sc_gather_body)
    xg = gather_fn(src_ids, x)

    yg = pl.pallas_call(
        _gmm_body,
        grid_spec=pltpu.PrefetchScalarGridSpec(
            num_scalar_prefetch=2,
            grid=(NJ, NBLK),
            in_specs=[
                pl.BlockSpec((BM, D_IN), lambda j, i, b_e, b_v: (i, 0)),
                pl.BlockSpec((1, D_IN, BN), lambda j, i, b_e, b_v: (b_e[i], 0, j)),
                pl.BlockSpec((1, 1, BN), lambda j, i, b_e, b_v: (b_e[i], 0, j)),
                pl.BlockSpec((BM, 1), lambda j, i, b_e, b_v: (i, 0)),
            ],
            out_specs=pl.BlockSpec((BM, BN), lambda j, i, b_e, b_v: (i, j)),
        ),
        out_shape=jax.ShapeDtypeStruct((PAD, D_OUT), jnp.float32),
    )(blk_e, blk_v, xg, We, be.reshape(E, 1, D_OUT), wsort.reshape(PAD, 1))

    combine_fn = functools.partial(
        pl.kernel,
        out_type=jax.ShapeDtypeStruct((N, D_OUT), jnp.float32),
        mesh=plsc.VectorSubcoreMesh(**_MESH),
        scratch_types=(
            [pltpu.VMEM((tpw,), jnp.int32)] * 2
            + [pltpu.VMEM((8, D_OUT), jnp.float32)] * 3
            + [pltpu.SemaphoreType.DMA] * 3
        ),
    )(_sc_combine_body)
    updates = combine_fn(d1f, d2f, yg)

    return (updates, gating)


# speedup vs baseline: 1.0498x; 1.0498x over previous
import functools

import jax
import jax.numpy as jnp
from jax import lax
from jax.experimental import pallas as pl
from jax.experimental.pallas import tpu as pltpu
from jax.experimental.pallas import tpu_sc as plsc

E = 8
TOPK = 2
N = 8192
D_IN = 4096
D_OUT = 4096
BM = 256
NB = N // BM
NBLK = N * TOPK // BM + E
PAD = NBLK * BM
BN = 1024
NJ = D_OUT // BN

_MESH = dict(core_axis_name="c", subcore_axis_name="s")
NC, NS = 2, 16
NW = NC * NS


def _top2(noisy):
    col = lax.broadcasted_iota(jnp.int32, noisy.shape, 1)
    m1 = jnp.max(noisy, axis=1, keepdims=True)
    a1 = jnp.min(jnp.where(noisy == m1, col, E), axis=1, keepdims=True)
    oh1 = col == a1
    masked = jnp.where(oh1, -jnp.inf, noisy)
    m2 = jnp.max(masked, axis=1, keepdims=True)
    a2 = jnp.min(jnp.where(masked == m2, col, E), axis=1, keepdims=True)
    oh2 = col == a2
    z = jnp.exp(m2 - m1)
    p1 = 1.0 / (1.0 + z)
    p2 = 1.0 - p1
    return oh1, oh2, p1, p2


def _router_body(x_ref, wg_ref, bg_ref, wn_ref, bn_ref, eps_ref,
                 gate_ref, d1_ref, d2_ref, w1_ref, w2_ref, meta_ref,
                 noisy_s, bc_s, cum_s, po_s):
    ph = pl.program_id(0)
    i = pl.program_id(1)

    @pl.when(ph == 0)
    def _phase0():
        xb = x_ref[...]
        logits = jnp.dot(xb, wg_ref[...], preferred_element_type=jnp.float32) + bg_ref[...]
        nl = jnp.dot(xb, wn_ref[...], preferred_element_type=jnp.float32) + bn_ref[...]
        sp = jnp.maximum(nl, 0.0) + jnp.log1p(jnp.exp(-jnp.abs(nl)))
        noisy = logits + eps_ref[...] * sp
        noisy_s[i] = noisy
        oh1, oh2, p1, p2 = _top2(noisy)
        gate_ref[...] = jnp.where(oh1, p1, 0.0) + jnp.where(oh2, p2, 0.0)
        d1_ref[...] = jnp.zeros_like(d1_ref)
        d2_ref[...] = jnp.zeros_like(d2_ref)
        w1_ref[...] = jnp.zeros_like(w1_ref)
        w2_ref[...] = jnp.zeros_like(w2_ref)
        sel = jnp.where(oh1, 1.0, 0.0) + jnp.where(oh2, 1.0, 0.0)
        bc_s[i] = jnp.sum(sel, axis=0, keepdims=True)

        @pl.when(i == 0)
        def _():
            meta_ref[...] = jnp.zeros_like(meta_ref)

    @pl.when(ph == 1)
    def _phase1():
        @pl.when(i == 0)
        def _prefix():
            bc = bc_s[...].reshape(NB, E)
            r32 = lax.broadcasted_iota(jnp.int32, (NB, NB), 0)
            c32 = lax.broadcasted_iota(jnp.int32, (NB, NB), 1)
            l32 = jnp.where(r32 > c32, 1.0, 0.0)
            cum_s[...] = jnp.dot(l32, bc, preferred_element_type=jnp.float32).reshape(NB, 1, E)
            tot = jnp.sum(bc, axis=0, keepdims=True)
            nb = jnp.floor((tot + (BM - 1.0)) / BM)
            r8 = lax.broadcasted_iota(jnp.int32, (E, E), 0)
            c8 = lax.broadcasted_iota(jnp.int32, (E, E), 1)
            u8 = jnp.where(r8 < c8, 1.0, 0.0)
            po_blk = jnp.dot(nb, u8, preferred_element_type=jnp.float32)
            po_s[...] = po_blk * BM
            bi = lax.broadcasted_iota(jnp.int32, (128, E), 0).astype(jnp.float32)
            cmat = jnp.where(bi >= po_blk, 1.0, 0.0)
            ones8 = jnp.ones((E, 1), jnp.float32)
            blk_e = jnp.dot(cmat, ones8, preferred_element_type=jnp.float32) - 1.0
            blk_e = jnp.clip(blk_e, 0.0, E - 1.0)
            totblk = jnp.dot(nb, ones8, preferred_element_type=jnp.float32)
            bi1 = lax.broadcasted_iota(jnp.int32, (128, 1), 0).astype(jnp.float32)
            validb = jnp.where(bi1 < totblk, 1.0, 0.0)
            zer = jnp.zeros((128, E - 2), jnp.float32)
            meta_ref[...] = jnp.concatenate([blk_e, validb, zer], axis=1).astype(jnp.int32)

        noisy = noisy_s[i]
        oh1, oh2, p1, p2 = _top2(noisy)
        gate_ref[...] = jnp.where(oh1, p1, 0.0) + jnp.where(oh2, p2, 0.0)
        sel = jnp.where(oh1, 1.0, 0.0) + jnp.where(oh2, 1.0, 0.0)
        rr = lax.broadcasted_iota(jnp.int32, (BM, BM), 0)
        cc = lax.broadcasted_iota(jnp.int32, (BM, BM), 1)
        ltri = jnp.where(rr > cc, 1.0, 0.0)
        rank = jnp.dot(ltri, sel, preferred_element_type=jnp.float32)
        dest_all = po_s[...] + cum_s[i] + rank
        d1_ref[...] = jnp.sum(jnp.where(oh1, dest_all, 0.0), axis=1, keepdims=True).astype(jnp.int32)
        d2_ref[...] = jnp.sum(jnp.where(oh2, dest_all, 0.0), axis=1, keepdims=True).astype(jnp.int32)
        w1_ref[...] = p1
        w2_ref[...] = p2


def _sc_scatter_body(d1_h, d2_h, w1f_h, w2f_h, tok_h, src_h, ws_h,
                     idx1, idx2, tokv, w1, w2, sem):
    wid = lax.axis_index("s") * NC + lax.axis_index("c")
    tpw = N // NW
    base = wid * tpw
    pltpu.sync_copy(d1_h.at[pl.ds(base, tpw)], idx1)
    pltpu.sync_copy(d2_h.at[pl.ds(base, tpw)], idx2)
    pltpu.sync_copy(w1f_h.at[pl.ds(base, tpw)], w1)
    pltpu.sync_copy(w2f_h.at[pl.ds(base, tpw)], w2)
    pltpu.sync_copy(tok_h.at[pl.ds(base, tpw)], tokv)
    h1 = pltpu.async_copy(tokv, src_h.at[idx1], sem)
    h2 = pltpu.async_copy(tokv, src_h.at[idx2], sem)
    h3 = pltpu.async_copy(w1, ws_h.at[idx1], sem)
    h4 = pltpu.async_copy(w2, ws_h.at[idx2], sem)
    h1.wait()
    h2.wait()
    h3.wait()
    h4.wait()


def _sc_gather_body(src_h, x_h, xg_h, idx_v, bufs0, bufs1, bufs2,
                    gs0, gs1, gs2, ws0, ws1, ws2):
    wid = lax.axis_index("s") * NC + lax.axis_index("c")
    spw = src_h.shape[0] // NW
    base = wid * spw
    bufs = (bufs0, bufs1, bufs2)
    gsem = (gs0, gs1, gs2)
    wsem = (ws0, ws1, ws2)
    pltpu.sync_copy(src_h.at[pl.ds(base, spw)], idx_v)

    def clamp(c, _):
        v = idx_v[pl.ds(c * 16, 16)]
        idx_v[pl.ds(c * 16, 16)] = jnp.minimum(jnp.maximum(v, 0), N - 1)
        return 0

    lax.fori_loop(0, spw // 16, clamp, 0)

    nit = spw // 24

    def gbody(it, _):
        @pl.when(it > 0)
        def _():
            for k in range(3):
                pltpu.make_async_copy(
                    bufs[k], xg_h.at[pl.ds(base + ((it - 1) * 3 + k) * 8, 8)],
                    wsem[k]).wait()

        hs = [pltpu.async_copy(x_h.at[idx_v.at[pl.ds((it * 3 + k) * 8, 8)]],
                               bufs[k], gsem[k]) for k in range(3)]
        for k in range(3):
            hs[k].wait()
            pltpu.async_copy(bufs[k], xg_h.at[pl.ds(base + (it * 3 + k) * 8, 8)],
                             wsem[k])
        return 0

    lax.fori_loop(0, nit, gbody, 0)
    for k in range(3):
        pltpu.make_async_copy(
            bufs[k], xg_h.at[pl.ds(base + ((nit - 1) * 3 + k) * 8, 8)],
            wsem[k]).wait()


def _gmm_body(be_s, bv_s, xg_ref, we_ref, bias_ref, w_ref, out_ref):
    i = pl.program_id(1)

    @pl.when(bv_s[i] == 1)
    def _():
        y = jnp.dot(xg_ref[...], we_ref[0], preferred_element_type=jnp.float32)
        out_ref[...] = (y + bias_ref[0]) * w_ref[...]


def _gmm_body_hi(be_s, bv_s, xg_ref, we_ref, bias_ref, w_ref, yg_in_ref,
                 out_ref):
    i = pl.program_id(1)

    @pl.when(bv_s[i] == 1)
    def _():
        y = jnp.dot(xg_ref[...], we_ref[0], preferred_element_type=jnp.float32)
        out_ref[...] = (y + bias_ref[0]) * w_ref[...]


def _sc_combine_body(d1_h, d2_h, yg_h, out_h, d1, d2,
                     bufa0, bufa1, bufb, sa0, sa1, sb):
    wid = lax.axis_index("s") * NC + lax.axis_index("c")
    tpw = N // NW
    base = wid * tpw
    pltpu.sync_copy(d1_h.at[pl.ds(base, tpw)], d1)
    pltpu.sync_copy(d2_h.at[pl.ds(base, tpw)], d2)

    nchunk = tpw // 8

    def start1(c, ba, sa):
        pltpu.async_copy(yg_h.at[d1.at[pl.ds(c * 8, 8)]], ba, sa)

    def start2(c):
        pltpu.async_copy(yg_h.at[d2.at[pl.ds(c * 8, 8)]], bufb, sb)

    def finish(c, ba, sa):
        pltpu.make_async_copy(yg_h.at[d1.at[pl.ds(c * 8, 8)]], ba, sa).wait()
        pltpu.make_async_copy(yg_h.at[d2.at[pl.ds(c * 8, 8)]], bufb, sb).wait()
        for r in range(8):
            def add16(t, _, r=r):
                for u in range(16):
                    o = t * 256 + u * 16
                    ba[r, pl.ds(o, 16)] = ba[r, pl.ds(o, 16)] + bufb[r, pl.ds(o, 16)]
                return 0

            lax.fori_loop(0, D_OUT // 256, add16, 0)
        pltpu.sync_copy(ba, out_h.at[pl.ds(base + c * 8, 8)])

    start1(0, bufa0, sa0)
    start2(0)

    def cbody(it, _):
        c0 = it * 2
        start1(c0 + 1, bufa1, sa1)
        finish(c0, bufa0, sa0)
        start2(c0 + 1)

        @pl.when(c0 + 2 < nchunk)
        def _():
            start1(c0 + 2, bufa0, sa0)

        finish(c0 + 1, bufa1, sa1)

        @pl.when(c0 + 2 < nchunk)
        def _():
            start2(c0 + 2)

        return 0

    lax.fori_loop(0, nchunk // 2, cbody, 0)


def kernel(x, Wg, bg, Wn, bn, We, be):
    eps = jax.random.normal(jax.random.key(42), (N, E), jnp.float32)
    tok = jnp.arange(N, dtype=jnp.int32)

    gating, d1a, d2a, w1a, w2a, meta = pl.pallas_call(
        _router_body,
        grid=(2, NB),
        in_specs=[
            pl.BlockSpec((BM, D_IN), lambda ph, i: (i * (1 - ph), 0)),
            pl.BlockSpec((D_IN, E), lambda ph, i: (0, 0)),
            pl.BlockSpec((1, E), lambda ph, i: (0, 0)),
            pl.BlockSpec((D_IN, E), lambda ph, i: (0, 0)),
            pl.BlockSpec((1, E), lambda ph, i: (0, 0)),
            pl.BlockSpec((BM, E), lambda ph, i: (i * (1 - ph), 0)),
        ],
        out_specs=[
            pl.BlockSpec((BM, E), lambda ph, i: (i, 0)),
            pl.BlockSpec((BM, 1), lambda ph, i: (i, 0)),
            pl.BlockSpec((BM, 1), lambda ph, i: (i, 0)),
            pl.BlockSpec((BM, 1), lambda ph, i: (i, 0)),
            pl.BlockSpec((BM, 1), lambda ph, i: (i, 0)),
            pl.BlockSpec((128, E), lambda ph, i: (0, 0)),
        ],
        out_shape=[
            jax.ShapeDtypeStruct((N, E), jnp.float32),
            jax.ShapeDtypeStruct((N, 1), jnp.int32),
            jax.ShapeDtypeStruct((N, 1), jnp.int32),
            jax.ShapeDtypeStruct((N, 1), jnp.float32),
            jax.ShapeDtypeStruct((N, 1), jnp.float32),
            jax.ShapeDtypeStruct((128, E), jnp.int32),
        ],
        scratch_shapes=[
            pltpu.VMEM((NB, BM, E), jnp.float32),
            pltpu.VMEM((NB, 1, E), jnp.float32),
            pltpu.VMEM((NB, 1, E), jnp.float32),
            pltpu.VMEM((1, E), jnp.float32),
        ],
    )(x, Wg, bg.reshape(1, E), Wn, bn.reshape(1, E), eps)

    blk_e = meta[:NBLK, 0]
    blk_v = meta[:NBLK, 1]
    d1f = d1a.reshape(N)
    d2f = d2a.reshape(N)

    tpw = N // NW
    scatter_fn = functools.partial(
        pl.kernel,
        out_type=[
            jax.ShapeDtypeStruct((PAD,), jnp.int32),
            jax.ShapeDtypeStruct((PAD,), jnp.float32),
        ],
        mesh=plsc.VectorSubcoreMesh(**_MESH),
        scratch_types=[
            pltpu.VMEM((tpw,), jnp.int32),
            pltpu.VMEM((tpw,), jnp.int32),
            pltpu.VMEM((tpw,), jnp.int32),
            pltpu.VMEM((tpw,), jnp.float32),
            pltpu.VMEM((tpw,), jnp.float32),
            pltpu.SemaphoreType.DMA,
        ],
    )(_sc_scatter_body)
    src_ids, wsort = scatter_fn(d1f, d2f, w1a.reshape(N), w2a.reshape(N), tok)

    half = PAD // 2
    nhb = NBLK // 2
    gather_fn = functools.partial(
        pl.kernel,
        out_type=jax.ShapeDtypeStruct((half, D_IN), jnp.float32),
        mesh=plsc.VectorSubcoreMesh(**_MESH),
        scratch_types=(
            [pltpu.VMEM((half // NW,), jnp.int32)]
            + [pltpu.VMEM((8, D_IN), jnp.float32)] * 3
            + [pltpu.SemaphoreType.DMA] * 6
        ),
    )(_sc_gather_body)
    xg_lo = gather_fn(src_ids[:half], x)
    xg_hi = gather_fn(src_ids[half:], x)

    ws2 = wsort.reshape(PAD, 1)
    be3 = be.reshape(E, 1, D_OUT)
    yg_lo = pl.pallas_call(
        _gmm_body,
        grid_spec=pltpu.PrefetchScalarGridSpec(
            num_scalar_prefetch=2,
            grid=(NJ, nhb),
            in_specs=[
                pl.BlockSpec((BM, D_IN), lambda j, i, b_e, b_v: (i, 0)),
                pl.BlockSpec((1, D_IN, BN), lambda j, i, b_e, b_v: (b_e[i], 0, j)),
                pl.BlockSpec((1, 1, BN), lambda j, i, b_e, b_v: (b_e[i], 0, j)),
                pl.BlockSpec((BM, 1), lambda j, i, b_e, b_v: (i, 0)),
            ],
            out_specs=pl.BlockSpec((BM, BN), lambda j, i, b_e, b_v: (i, j)),
        ),
        out_shape=jax.ShapeDtypeStruct((PAD, D_OUT), jnp.float32),
    )(blk_e[:nhb], blk_v[:nhb], xg_lo, We, be3, ws2[:half])

    yg = pl.pallas_call(
        _gmm_body_hi,
        grid_spec=pltpu.PrefetchScalarGridSpec(
            num_scalar_prefetch=2,
            grid=(NJ, nhb),
            in_specs=[
                pl.BlockSpec((BM, D_IN), lambda j, i, b_e, b_v: (i, 0)),
                pl.BlockSpec((1, D_IN, BN), lambda j, i, b_e, b_v: (b_e[i], 0, j)),
                pl.BlockSpec((1, 1, BN), lambda j, i, b_e, b_v: (b_e[i], 0, j)),
                pl.BlockSpec((BM, 1), lambda j, i, b_e, b_v: (i, 0)),
                pl.BlockSpec(memory_space=pl.ANY),
            ],
            out_specs=pl.BlockSpec((BM, BN),
                                   lambda j, i, b_e, b_v: (i + NBLK // 2, j)),
        ),
        out_shape=jax.ShapeDtypeStruct((PAD, D_OUT), jnp.float32),
        input_output_aliases={6: 0},
    )(blk_e[nhb:], blk_v[nhb:], xg_hi, We, be3, ws2[half:], yg_lo)

    combine_fn = functools.partial(
        pl.kernel,
        out_type=jax.ShapeDtypeStruct((N, D_OUT), jnp.float32),
        mesh=plsc.VectorSubcoreMesh(**_MESH),
        scratch_types=(
            [pltpu.VMEM((tpw,), jnp.int32)] * 2
            + [pltpu.VMEM((8, D_OUT), jnp.float32)] * 3
            + [pltpu.SemaphoreType.DMA] * 3
        ),
    )(_sc_combine_body)
    updates = combine_fn(d1f, d2f, yg)

    return (updates, gating)
